# baseline (speedup 1.0000x reference)
import functools

import jax
import jax.numpy as jnp
from jax import lax
from jax.experimental import pallas as pl
from jax.experimental.pallas import tpu as pltpu
from jax.experimental.pallas import tpu_sc as plsc

_PAD = 128


def _l1_body(hid, e_ref, w1_ref, b1_ref, o_ref):
    h = jnp.dot(e_ref[...], w1_ref[...], preferred_element_type=jnp.float32)
    o_ref[:, :hid] = jnp.maximum(h + b1_ref[...], 0.0)
    o_ref[:, hid:] = jnp.zeros((o_ref.shape[0], _PAD - hid), jnp.float32)


def _layer1(esm, W1, b1, row_block):
    g, d = esm.shape
    hid = W1.shape[1]
    grid = (g + row_block - 1) // row_block
    return pl.pallas_call(
        functools.partial(_l1_body, hid),
        grid=(grid,),
        in_specs=[
            pl.BlockSpec((row_block, d), lambda i: (i, 0)),
            pl.BlockSpec((d, hid), lambda i: (0, 0)),
            pl.BlockSpec((1, hid), lambda i: (0, 0)),
        ],
        out_specs=pl.BlockSpec((row_block, _PAD), lambda i: (i, 0)),
        out_shape=jax.ShapeDtypeStruct((g, _PAD), jnp.float32),
    )(esm, W1, b1.reshape(1, hid))


def _gather_body(n_chunks, ch, nc, hid, idx_hbm, tbl_hbm, out_hbm,
                 idx_v, r0, r1, pk, s0, s1):
    wid = lax.axis_index("s") * nc + lax.axis_index("c")
    per_w = n_chunks * ch
    base = wid * per_w
    chp = ch // 2
    pltpu.sync_copy(idx_hbm.at[pl.ds(base, per_w)], idx_v)
    bufs, sems, descs = (r0, r1), (s0, s1), [None, None]

    def pack_store(j):
        src = bufs[j % 2]

        def body(f, _):
            row_a = jnp.full((16,), 2 * f, jnp.int32)
            row_b = row_a + 1
            row_p = jnp.full((16,), f, jnp.int32)
            for v in range(hid // 16):
                lanes = lax.iota(jnp.int32, 16) + v * 16
                plsc.store_scatter(pk, [row_p, lanes],
                                   plsc.load_gather(src, [row_a, lanes]))
                plsc.store_scatter(pk, [row_p, lanes + hid],
                                   plsc.load_gather(src, [row_b, lanes]))
            return 0

        lax.fori_loop(0, chp, body, 0)
        pltpu.sync_copy(pk, out_hbm.at[pl.ds(wid * (n_chunks * chp) + j * chp,
                                             chp)])

    descs[0] = pltpu.async_copy(tbl_hbm.at[idx_v.at[pl.ds(0, ch)]], r0, s0)
    for i in range(1, n_chunks):
        descs[i % 2] = pltpu.async_copy(
            tbl_hbm.at[idx_v.at[pl.ds(i * ch, ch)]], bufs[i % 2], sems[i % 2])
        descs[(i - 1) % 2].wait()
        pack_store(i - 1)
    descs[(n_chunks - 1) % 2].wait()
    pack_store(n_chunks - 1)


def _sc_gather_packed(table, idx_flat, hid):
    bs = idx_flat.shape[0]
    info = plsc.get_sparse_core_info()
    nc, ns = info.num_cores, info.num_subcores
    nw = nc * ns
    per_w = bs // nw
    ch = 320
    n_chunks = per_w // ch
    body = functools.partial(_gather_body, n_chunks, ch, nc, hid)
    kern = pl.kernel(
        body,
        out_type=jax.ShapeDtypeStruct((bs // 2, _PAD), jnp.float32),
        mesh=plsc.VectorSubcoreMesh(core_axis_name="c", subcore_axis_name="s"),
        scratch_types=[
            pltpu.VMEM((per_w,), jnp.int32),
            pltpu.VMEM((ch, _PAD), jnp.float32),
            pltpu.VMEM((ch, _PAD), jnp.float32),
            pltpu.VMEM((ch // 2, _PAD), jnp.float32),
            pltpu.SemaphoreType.DMA,
            pltpu.SemaphoreType.DMA,
        ],
        compiler_params=pltpu.CompilerParams(use_tc_tiling_on_sc=True, needs_layout_passes=False),
    )
    return kern(idx_flat, table)


def _l2_body(bh, x_ref, w2tlo_ref, w2thi_ref, b2_ref, *rest):
    o_ref = rest[-1]
    x = x_ref[0]
    y_lo = lax.dot_general(
        w2tlo_ref[...], x, dimension_numbers=(((1,), (1,)), ((), ())),
        preferred_element_type=jnp.float32)
    y_hi = lax.dot_general(
        w2thi_ref[...], x, dimension_numbers=(((1,), (1,)), ((), ())),
        preferred_element_type=jnp.float32)
    o_ref[0, :, :bh] = y_lo + b2_ref[...]
    o_ref[0, :, bh:] = y_hi + b2_ref[...]


def _layer2_t(rows3, W2pTlo, W2pThi, b2, s_slab, s0, s_total, b, hid, prev):
    bh = b // 2
    in_specs = [
        pl.BlockSpec((1, bh, _PAD), lambda i: (i, 0, 0)),
        pl.BlockSpec((hid, _PAD), lambda i: (0, 0)),
        pl.BlockSpec((hid, _PAD), lambda i: (0, 0)),
        pl.BlockSpec((hid, 1), lambda i: (0, 0)),
    ]
    args = [rows3, W2pTlo, W2pThi, b2.reshape(hid, 1)]
    kwargs = {}
    if prev is not None:
        in_specs.append(pl.BlockSpec(memory_space=pl.ANY))
        args.append(prev)
        kwargs["input_output_aliases"] = {4: 0}
    return pl.pallas_call(
        functools.partial(_l2_body, bh),
        grid=(s_slab,),
        in_specs=in_specs,
        out_specs=pl.BlockSpec((1, hid, b), lambda i, s0=s0: (i + s0, 0, 0)),
        out_shape=jax.ShapeDtypeStruct((s_total, hid, b), jnp.float32),
        **kwargs,
    )(*args)


def kernel(pert_esm_indices, esm_embeddings, W1, b1, W2, b2):
    idx = pert_esm_indices
    if idx.shape[-1] == 1:
        idx = jnp.squeeze(idx, axis=-1)
    b, s = idx.shape
    d, hid = W1.shape
    bh = b // 2
    W2pTlo = jnp.pad(W2, ((0, _PAD - hid), (0, 0))).T
    W2pThi = jnp.pad(W2, ((_PAD - hid, 0), (0, 0))).T
    table = _layer1(esm_embeddings, W1, b1, row_block=2000)
    idx_p = idx.T.astype(jnp.int32).reshape(s, 2, bh).transpose(0, 2, 1)

    n_slabs = 2
    s_slab = s // n_slabs
    out_t = None
    slabs = []
    for k in range(n_slabs):
        idx_k = lax.slice_in_dim(idx_p, k * s_slab, (k + 1) * s_slab).reshape(-1)
        slabs.append(
            _sc_gather_packed(table, idx_k, hid).reshape(s_slab, bh, _PAD))
    for k in range(n_slabs):
        out_t = _layer2_t(slabs[k], W2pTlo, W2pThi, b2, s_slab, k * s_slab,
                          s, b, hid, out_t)
    return jnp.transpose(out_t, (2, 0, 1))


# speedup vs baseline: 1.4711x; 1.4711x over previous
import functools

import jax
import jax.numpy as jnp
from jax import lax
from jax.experimental import pallas as pl
from jax.experimental.pallas import tpu as pltpu
from jax.experimental.pallas import tpu_sc as plsc

_PAD = 128


def _l1_body(hid, e_ref, w1_ref, b1_ref, o_ref):
    h = jnp.dot(e_ref[...], w1_ref[...], preferred_element_type=jnp.float32)
    o_ref[:, :hid] = jnp.maximum(h + b1_ref[...], 0.0)
    o_ref[:, hid:] = jnp.zeros((o_ref.shape[0], _PAD - hid), jnp.float32)


def _layer1(esm, W1, b1, row_block):
    g, d = esm.shape
    hid = W1.shape[1]
    grid = (g + row_block - 1) // row_block
    return pl.pallas_call(
        functools.partial(_l1_body, hid),
        grid=(grid,),
        in_specs=[
            pl.BlockSpec((row_block, d), lambda i: (i, 0)),
            pl.BlockSpec((d, hid), lambda i: (0, 0)),
            pl.BlockSpec((1, hid), lambda i: (0, 0)),
        ],
        out_specs=pl.BlockSpec((row_block, _PAD), lambda i: (i, 0)),
        out_shape=jax.ShapeDtypeStruct((g, _PAD), jnp.float32),
    )(esm, W1, b1.reshape(1, hid))


def _gather_body(n_chunks, ch, nc, idx_hbm, tbl_hbm, out_hbm,
                 idx_v, r0, r1, s0, s1):
    wid = lax.axis_index("s") * nc + lax.axis_index("c")
    per_w = n_chunks * ch
    base = wid * per_w
    pltpu.sync_copy(idx_hbm.at[pl.ds(base, per_w)], idx_v)
    bufs, sems, descs = (r0, r1), (s0, s1), [None, None]
    descs[0] = pltpu.async_copy(tbl_hbm.at[idx_v.at[pl.ds(0, ch)]], r0, s0)
    for i in range(1, n_chunks):
        descs[i % 2] = pltpu.async_copy(
            tbl_hbm.at[idx_v.at[pl.ds(i * ch, ch)]], bufs[i % 2], sems[i % 2])
        descs[(i - 1) % 2].wait()
        pltpu.sync_copy(bufs[(i - 1) % 2],
                        out_hbm.at[pl.ds(base + (i - 1) * ch, ch)])
    last = n_chunks - 1
    descs[last % 2].wait()
    pltpu.sync_copy(bufs[last % 2], out_hbm.at[pl.ds(base + last * ch, ch)])


def _sc_gather(table, idx_flat):
    _, hid = table.shape
    bs = idx_flat.shape[0]
    info = plsc.get_sparse_core_info()
    nc, ns = info.num_cores, info.num_subcores
    nw = nc * ns
    per_w = bs // nw
    ch = 256
    n_chunks = per_w // ch
    body = functools.partial(_gather_body, n_chunks, ch, nc)
    kern = pl.kernel(
        body,
        out_type=jax.ShapeDtypeStruct((bs, hid), jnp.float32),
        mesh=plsc.VectorSubcoreMesh(core_axis_name="c", subcore_axis_name="s"),
        scratch_types=[
            pltpu.VMEM((per_w,), jnp.int32),
            pltpu.VMEM((ch, hid), jnp.float32),
            pltpu.VMEM((ch, hid), jnp.float32),
            pltpu.SemaphoreType.DMA,
            pltpu.SemaphoreType.DMA,
        ],
        compiler_params=pltpu.CompilerParams(use_tc_tiling_on_sc=True),
    )
    return kern(idx_flat, table)


def _l2_body(x_ref, w2t_ref, b2_ref, *rest):
    o_ref = rest[-1]
    y = lax.dot_general(
        w2t_ref[...], x_ref[0],
        dimension_numbers=(((1,), (1,)), ((), ())),
        preferred_element_type=jnp.float32,
    )
    o_ref[0] = y + b2_ref[...]


def _layer2_t(rows3, W2pT, b2, s_slab, s0, s_total, b, hid, prev):
    in_specs = [
        pl.BlockSpec((1, b, _PAD), lambda i: (i, 0, 0)),
        pl.BlockSpec((hid, _PAD), lambda i: (0, 0)),
        pl.BlockSpec((hid, 1), lambda i: (0, 0)),
    ]
    args = [rows3, W2pT, b2.reshape(hid, 1)]
    kwargs = {}
    if prev is not None:
        in_specs.append(pl.BlockSpec(memory_space=pl.ANY))
        args.append(prev)
        kwargs["input_output_aliases"] = {3: 0}
    return pl.pallas_call(
        _l2_body,
        grid=(s_slab,),
        in_specs=in_specs,
        out_specs=pl.BlockSpec((1, hid, b), lambda i, s0=s0: (i + s0, 0, 0)),
        out_shape=jax.ShapeDtypeStruct((s_total, hid, b), jnp.float32),
        **kwargs,
    )(*args)


def kernel(pert_esm_indices, esm_embeddings, W1, b1, W2, b2):
    idx = pert_esm_indices
    if idx.shape[-1] == 1:
        idx = jnp.squeeze(idx, axis=-1)
    b, s = idx.shape
    d, hid = W1.shape
    W2pT = jnp.pad(W2, ((0, _PAD - hid), (0, 0))).T
    table = _layer1(esm_embeddings, W1, b1, row_block=2000)
    idx_t = idx.T.astype(jnp.int32)

    n_slabs = 2
    s_slab = s // n_slabs
    out_t = None
    slabs = []
    for k in range(n_slabs):
        idx_k = lax.slice_in_dim(idx_t, k * s_slab, (k + 1) * s_slab).reshape(-1)
        slabs.append(_sc_gather(table, idx_k).reshape(s_slab, b, _PAD))
    for k in range(n_slabs):
        out_t = _layer2_t(slabs[k], W2pT, b2, s_slab, k * s_slab, s, b, hid,
                          out_t)
    return jnp.transpose(out_t, (2, 0, 1))


# speedup vs baseline: 1.4779x; 1.0046x over previous
import functools

import jax
import jax.numpy as jnp
from jax import lax
from jax.experimental import pallas as pl
from jax.experimental.pallas import tpu as pltpu
from jax.experimental.pallas import tpu_sc as plsc

_PAD = 128


def _l1_body(hid, e_ref, w1_ref, b1_ref, o_ref):
    h = jnp.dot(e_ref[...], w1_ref[...], preferred_element_type=jnp.float32)
    o_ref[:, :hid] = jnp.maximum(h + b1_ref[...], 0.0)
    o_ref[:, hid:] = jnp.zeros((o_ref.shape[0], _PAD - hid), jnp.float32)


def _layer1(esm, W1, b1, row_block):
    g, d = esm.shape
    hid = W1.shape[1]
    grid = (g + row_block - 1) // row_block
    return pl.pallas_call(
        functools.partial(_l1_body, hid),
        grid=(grid,),
        in_specs=[
            pl.BlockSpec((row_block, d), lambda i: (i, 0)),
            pl.BlockSpec((d, hid), lambda i: (0, 0)),
            pl.BlockSpec((1, hid), lambda i: (0, 0)),
        ],
        out_specs=pl.BlockSpec((row_block, _PAD), lambda i: (i, 0)),
        out_shape=jax.ShapeDtypeStruct((g, _PAD), jnp.float32),
    )(esm, W1, b1.reshape(1, hid))


def _gather_body(n_chunks, ch, nc, idx_hbm, tbl_hbm, out_hbm,
                 idx_v, r0, r1, r2, g0, g1, g2, t0, t1, t2):
    wid = lax.axis_index("s") * nc + lax.axis_index("c")
    per_w = n_chunks * ch
    base = wid * per_w
    pltpu.sync_copy(idx_hbm.at[pl.ds(base, per_w)], idx_v)
    bufs = (r0, r1, r2)
    gsem = (g0, g1, g2)
    ssem = (t0, t1, t2)
    gd = [None] * n_chunks
    sd = [None] * 3

    def gather(i):
        return pltpu.async_copy(
            tbl_hbm.at[idx_v.at[pl.ds(i * ch, ch)]], bufs[i % 3], gsem[i % 3])

    gd[0] = gather(0)
    if n_chunks > 1:
        gd[1] = gather(1)
    for i in range(n_chunks):
        gd[i].wait()
        if i + 2 < n_chunks:
            if sd[(i + 2) % 3] is not None:
                sd[(i + 2) % 3].wait()
                sd[(i + 2) % 3] = None
            gd[i + 2] = gather(i + 2)
        sd[i % 3] = pltpu.async_copy(
            bufs[i % 3], out_hbm.at[pl.ds(base + i * ch, ch)], ssem[i % 3])
    for k in range(3):
        if sd[k] is not None:
            sd[k].wait()


def _sc_gather(table, idx_flat):
    _, hid = table.shape
    bs = idx_flat.shape[0]
    info = plsc.get_sparse_core_info()
    nc, ns = info.num_cores, info.num_subcores
    nw = nc * ns
    per_w = bs // nw
    ch = 256
    n_chunks = per_w // ch
    body = functools.partial(_gather_body, n_chunks, ch, nc)
    kern = pl.kernel(
        body,
        out_type=jax.ShapeDtypeStruct((bs, hid), jnp.float32),
        mesh=plsc.VectorSubcoreMesh(core_axis_name="c", subcore_axis_name="s"),
        scratch_types=[
            pltpu.VMEM((per_w,), jnp.int32),
            pltpu.VMEM((ch, hid), jnp.float32),
            pltpu.VMEM((ch, hid), jnp.float32),
            pltpu.VMEM((ch, hid), jnp.float32),
            pltpu.SemaphoreType.DMA,
            pltpu.SemaphoreType.DMA,
            pltpu.SemaphoreType.DMA,
            pltpu.SemaphoreType.DMA,
            pltpu.SemaphoreType.DMA,
            pltpu.SemaphoreType.DMA,
        ],
        compiler_params=pltpu.CompilerParams(use_tc_tiling_on_sc=True),
    )
    return kern(idx_flat, table)


def _l2_body(x_ref, w2t_ref, b2_ref, *rest):
    o_ref = rest[-1]
    y = lax.dot_general(
        w2t_ref[...], x_ref[0],
        dimension_numbers=(((1,), (1,)), ((), ())),
        preferred_element_type=jnp.float32,
    )
    o_ref[0] = y + b2_ref[...]


def _layer2_t(rows3, W2pT, b2, s_slab, s0, s_total, b, hid, prev):
    in_specs = [
        pl.BlockSpec((1, b, _PAD), lambda i: (i, 0, 0)),
        pl.BlockSpec((hid, _PAD), lambda i: (0, 0)),
        pl.BlockSpec((hid, 1), lambda i: (0, 0)),
    ]
    args = [rows3, W2pT, b2.reshape(hid, 1)]
    kwargs = {}
    if prev is not None:
        in_specs.append(pl.BlockSpec(memory_space=pl.ANY))
        args.append(prev)
        kwargs["input_output_aliases"] = {3: 0}
    return pl.pallas_call(
        _l2_body,
        grid=(s_slab,),
        in_specs=in_specs,
        out_specs=pl.BlockSpec((1, hid, b), lambda i, s0=s0: (i + s0, 0, 0)),
        out_shape=jax.ShapeDtypeStruct((s_total, hid, b), jnp.float32),
        **kwargs,
    )(*args)


def kernel(pert_esm_indices, esm_embeddings, W1, b1, W2, b2):
    idx = pert_esm_indices
    if idx.shape[-1] == 1:
        idx = jnp.squeeze(idx, axis=-1)
    b, s = idx.shape
    d, hid = W1.shape
    W2pT = jnp.pad(W2, ((0, _PAD - hid), (0, 0))).T
    table = _layer1(esm_embeddings, W1, b1, row_block=2000)
    idx_t = idx.T.astype(jnp.int32)

    n_slabs = 2
    s_slab = s // n_slabs
    out_t = None
    slabs = []
    for k in range(n_slabs):
        idx_k = lax.slice_in_dim(idx_t, k * s_slab, (k + 1) * s_slab).reshape(-1)
        slabs.append(_sc_gather(table, idx_k).reshape(s_slab, b, _PAD))
    for k in range(n_slabs):
        out_t = _layer2_t(slabs[k], W2pT, b2, s_slab, k * s_slab, s, b, hid,
                          out_t)
    return jnp.transpose(out_t, (2, 0, 1))


# speedup vs baseline: 1.4903x; 1.0083x over previous
import functools

import jax
import jax.numpy as jnp
from jax import lax
from jax.experimental import pallas as pl
from jax.experimental.pallas import tpu as pltpu
from jax.experimental.pallas import tpu_sc as plsc

_PAD = 128


def _l1_body(hid, e_ref, w1_ref, b1_ref, o_ref):
    h = jnp.dot(e_ref[...], w1_ref[...], preferred_element_type=jnp.float32)
    o_ref[:, :hid] = jnp.maximum(h + b1_ref[...], 0.0)
    o_ref[:, hid:] = jnp.zeros((o_ref.shape[0], _PAD - hid), jnp.float32)


def _layer1(esm, W1, b1, row_block):
    g, d = esm.shape
    hid = W1.shape[1]
    grid = (g + row_block - 1) // row_block
    return pl.pallas_call(
        functools.partial(_l1_body, hid),
        grid=(grid,),
        in_specs=[
            pl.BlockSpec((row_block, d), lambda i: (i, 0)),
            pl.BlockSpec((d, hid), lambda i: (0, 0)),
            pl.BlockSpec((1, hid), lambda i: (0, 0)),
        ],
        out_specs=pl.BlockSpec((row_block, _PAD), lambda i: (i, 0)),
        out_shape=jax.ShapeDtypeStruct((g, _PAD), jnp.float32),
    )(esm, W1, b1.reshape(1, hid))


def _gather_body(n_chunks, ch, nc, idx_hbm, tbl_hbm, out_hbm,
                 idx_v, r0, r1, r2, g0, g1, g2, t0, t1, t2):
    wid = lax.axis_index("s") * nc + lax.axis_index("c")
    per_w = n_chunks * ch
    base = wid * per_w
    pltpu.sync_copy(idx_hbm.at[pl.ds(base, per_w)], idx_v)
    bufs = (r0, r1, r2)
    gsem = (g0, g1, g2)
    ssem = (t0, t1, t2)
    gd = [None] * n_chunks
    sd = [None] * 3

    def gather(i):
        return pltpu.async_copy(
            tbl_hbm.at[idx_v.at[pl.ds(i * ch, ch)]], bufs[i % 3], gsem[i % 3])

    gd[0] = gather(0)
    if n_chunks > 1:
        gd[1] = gather(1)
    for i in range(n_chunks):
        gd[i].wait()
        if i + 2 < n_chunks:
            if sd[(i + 2) % 3] is not None:
                sd[(i + 2) % 3].wait()
                sd[(i + 2) % 3] = None
            gd[i + 2] = gather(i + 2)
        sd[i % 3] = pltpu.async_copy(
            bufs[i % 3], out_hbm.at[pl.ds(base + i * ch, ch)], ssem[i % 3])
    for k in range(3):
        if sd[k] is not None:
            sd[k].wait()


def _sc_gather(table, idx_flat):
    _, hid = table.shape
    bs = idx_flat.shape[0]
    info = plsc.get_sparse_core_info()
    nc, ns = info.num_cores, info.num_subcores
    nw = nc * ns
    per_w = bs // nw
    ch = 320
    n_chunks = per_w // ch
    body = functools.partial(_gather_body, n_chunks, ch, nc)
    kern = pl.kernel(
        body,
        out_type=jax.ShapeDtypeStruct((bs, hid), jnp.float32),
        mesh=plsc.VectorSubcoreMesh(core_axis_name="c", subcore_axis_name="s"),
        scratch_types=[
            pltpu.VMEM((per_w,), jnp.int32),
            pltpu.VMEM((ch, hid), jnp.float32),
            pltpu.VMEM((ch, hid), jnp.float32),
            pltpu.VMEM((ch, hid), jnp.float32),
            pltpu.SemaphoreType.DMA,
            pltpu.SemaphoreType.DMA,
            pltpu.SemaphoreType.DMA,
            pltpu.SemaphoreType.DMA,
            pltpu.SemaphoreType.DMA,
            pltpu.SemaphoreType.DMA,
        ],
        compiler_params=pltpu.CompilerParams(use_tc_tiling_on_sc=True),
    )
    return kern(idx_flat, table)


def _l2_body(x_ref, w2t_ref, b2_ref, *rest):
    o_ref = rest[-1]
    y = lax.dot_general(
        w2t_ref[...], x_ref[0],
        dimension_numbers=(((1,), (1,)), ((), ())),
        preferred_element_type=jnp.float32,
    )
    o_ref[0] = y + b2_ref[...]


def _layer2_t(rows3, W2pT, b2, s_slab, s0, s_total, b, hid, prev):
    in_specs = [
        pl.BlockSpec((1, b, _PAD), lambda i: (i, 0, 0)),
        pl.BlockSpec((hid, _PAD), lambda i: (0, 0)),
        pl.BlockSpec((hid, 1), lambda i: (0, 0)),
    ]
    args = [rows3, W2pT, b2.reshape(hid, 1)]
    kwargs = {}
    if prev is not None:
        in_specs.append(pl.BlockSpec(memory_space=pl.ANY))
        args.append(prev)
        kwargs["input_output_aliases"] = {3: 0}
    return pl.pallas_call(
        _l2_body,
        grid=(s_slab,),
        in_specs=in_specs,
        out_specs=pl.BlockSpec((1, hid, b), lambda i, s0=s0: (i + s0, 0, 0)),
        out_shape=jax.ShapeDtypeStruct((s_total, hid, b), jnp.float32),
        **kwargs,
    )(*args)


def kernel(pert_esm_indices, esm_embeddings, W1, b1, W2, b2):
    idx = pert_esm_indices
    if idx.shape[-1] == 1:
        idx = jnp.squeeze(idx, axis=-1)
    b, s = idx.shape
    d, hid = W1.shape
    W2pT = jnp.pad(W2, ((0, _PAD - hid), (0, 0))).T
    table = _layer1(esm_embeddings, W1, b1, row_block=2000)
    idx_t = idx.T.astype(jnp.int32)

    n_slabs = 2
    s_slab = s // n_slabs
    out_t = None
    slabs = []
    for k in range(n_slabs):
        idx_k = lax.slice_in_dim(idx_t, k * s_slab, (k + 1) * s_slab).reshape(-1)
        slabs.append(_sc_gather(table, idx_k).reshape(s_slab, b, _PAD))
    for k in range(n_slabs):
        out_t = _layer2_t(slabs[k], W2pT, b2, s_slab, k * s_slab, s, b, hid,
                          out_t)
    return jnp.transpose(out_t, (2, 0, 1))
